# Initial kernel scaffold; baseline (speedup 1.0000x reference)
import jax
import jax.numpy as jnp
from jax.experimental import pallas as pl


def kernel(src_tokens, padded_coordinates, src_distance, src_edge_type, edge_index, params):
    raise NotImplementedError("write your pallas kernel here")


# speedup vs baseline: 1.4171x; 1.4171x over previous
import functools

import jax
import jax.numpy as jnp
from jax import lax
from jax.experimental import pallas as pl
from jax.experimental.pallas import tpu as pltpu
from jax.experimental.pallas import tpu_sc as plsc

DIM = 128
NSLICE = DIM // 16
CHUNK = 32
NTILES = 32
BLK = 512


def _sc_edge_body(nodes_per_tile, epad,
                  q_hbm, k_hbm, s_hbm, p_hbm, rows_hbm, cols_hbm, off_hbm,
                  aggs_hbm, aggv_hbm,
                  off_v, rows_v, cols_v, qbuf, kbuf, sbuf, pbuf,
                  accs, accv, sem):
    wid = lax.axis_index("s") * 2 + lax.axis_index("c")
    lane = lax.broadcasted_iota(jnp.int32, (16,), 0)
    words_per_tile = nodes_per_tile * DIM

    pltpu.sync_copy(off_hbm, off_v)

    lo = off_v[pl.ds(wid, 16)][0]
    hi = off_v[pl.ds(wid + 1, 16)][0]
    base_node = wid * nodes_per_tile
    start = jnp.bitwise_and(lo, jnp.int32(~7))
    nch = (hi - start + (CHUNK - 1)) // CHUNK

    zeros16 = jnp.zeros((16,), jnp.float32)

    def zbody(t, c):
        accs[pl.ds(t * 16, 16)] = zeros16
        accv[pl.ds(t * 16, 16)] = zeros16
        return c

    lax.fori_loop(0, words_per_tile // 16, zbody, 0)

    def cbody(g, c):
        e0 = pl.multiple_of(start + g * CHUNK, 8)
        pltpu.sync_copy(rows_hbm.at[pl.ds(e0, CHUNK)], rows_v)
        pltpu.sync_copy(cols_hbm.at[pl.ds(e0, CHUNK)], cols_v)
        cps = [pltpu.async_copy(q_hbm.at[rows_v], qbuf, sem),
               pltpu.async_copy(k_hbm.at[cols_v], kbuf, sem),
               pltpu.async_copy(s_hbm.at[cols_v], sbuf, sem),
               pltpu.async_copy(p_hbm.at[cols_v], pbuf, sem)]
        for cp in cps:
            cp.wait()
        for h in range(CHUNK // 16):
            rows16 = rows_v[pl.ds(16 * h, 16)]
            for e in range(16):
                ei = 16 * h + e
                eid = e0 + ei
                valid = jnp.logical_and(eid >= lo, eid < hi)
                dot = qbuf[ei, pl.ds(0, 16)] * kbuf[ei, pl.ds(0, 16)]
                for j in range(1, NSLICE):
                    dot = dot + (qbuf[ei, pl.ds(16 * j, 16)]
                                 * kbuf[ei, pl.ds(16 * j, 16)])
                for sh in (8, 4, 2, 1):
                    dot = dot + dot.at[lane ^ sh].get(
                        mode="promise_in_bounds")
                w = dot * jnp.where(valid, jnp.float32(1.0),
                                    jnp.float32(0.0))
                rb = jnp.clip(rows16[e] - base_node, 0,
                              nodes_per_tile - 1) * DIM
                for j in range(NSLICE):
                    plsc.addupdate(accs.at[pl.ds(rb + 16 * j, 16)],
                                   w * sbuf[ei, pl.ds(16 * j, 16)])
                    plsc.addupdate(accv.at[pl.ds(rb + 16 * j, 16)],
                                   w * pbuf[ei, pl.ds(16 * j, 16)])
        return c

    lax.fori_loop(0, nch, cbody, 0)

    pltpu.sync_copy(accs, aggs_hbm.at[pl.ds(wid * words_per_tile,
                                            words_per_tile)])
    pltpu.sync_copy(accv, aggv_hbm.at[pl.ds(wid * words_per_tile,
                                            words_per_tile)])


def _make_sc_edge(num_nodes, epad):
    nodes_per_tile = num_nodes // NTILES
    words_per_tile = nodes_per_tile * DIM
    mesh = plsc.VectorSubcoreMesh(core_axis_name="c", subcore_axis_name="s")
    return pl.kernel(
        functools.partial(_sc_edge_body, nodes_per_tile, epad),
        out_type=[jax.ShapeDtypeStruct((num_nodes * DIM,), jnp.float32),
                  jax.ShapeDtypeStruct((num_nodes * DIM,), jnp.float32)],
        mesh=mesh,
        scratch_types=[
            pltpu.VMEM((48,), jnp.int32),
            pltpu.VMEM((CHUNK,), jnp.int32),
            pltpu.VMEM((CHUNK,), jnp.int32),
            pltpu.VMEM((CHUNK, DIM), jnp.float32),
            pltpu.VMEM((CHUNK, DIM), jnp.float32),
            pltpu.VMEM((CHUNK, DIM), jnp.float32),
            pltpu.VMEM((CHUNK, DIM), jnp.float32),
            pltpu.VMEM((words_per_tile,), jnp.float32),
            pltpu.VMEM((words_per_tile,), jnp.float32),
            pltpu.SemaphoreType.DMA,
        ],
        name="sc_edge_agg",
    )


def _row_spec(nrow, ncol):
    return pl.BlockSpec((nrow, ncol), lambda i: (i, 0))


def _full_spec(shape):
    return pl.BlockSpec(shape, lambda i: tuple(0 for _ in shape))


def _proj(x, v, qwT, qb, kwT, kb, swT, sb, vvwT, vvb):
    q = jnp.dot(x, qwT, preferred_element_type=jnp.float32) + qb
    k = jnp.dot(x, kwT, preferred_element_type=jnp.float32) + kb
    vs = jnp.dot(x, swT, preferred_element_type=jnp.float32) + sb
    vv = jnp.dot(x, vvwT, preferred_element_type=jnp.float32) + vvb
    return q, k, vs, vv * v


def _tc_init_body(tok_ref, coord_ref, embed_ref, vwT_ref, vecb_ref,
                  qwT_ref, qb_ref, kwT_ref, kb_ref, swT_ref, sb_ref,
                  vvwT_ref, vvb_ref,
                  x_ref, v_ref, q_ref, k_ref, vs_ref, p_ref):
    tok = tok_ref[...]
    oh = (tok == lax.broadcasted_iota(jnp.int32, (1, DIM), 1)
          ).astype(jnp.float32)
    x = jnp.dot(oh, embed_ref[...], preferred_element_type=jnp.float32)
    c = coord_ref[...]
    vwT = vwT_ref[...]
    v = (vecb_ref[...] + c[:, 0:1] * vwT[0:1, :]
         + c[:, 1:2] * vwT[1:2, :] + c[:, 2:3] * vwT[2:3, :])
    q, k, vs, p = _proj(x, v, qwT_ref[...], qb_ref[...], kwT_ref[...],
                        kb_ref[...], swT_ref[...], sb_ref[...],
                        vvwT_ref[...], vvb_ref[...])
    x_ref[...], v_ref[...] = x, v
    q_ref[...], k_ref[...], vs_ref[...], p_ref[...] = q, k, vs, p


def _tc_mid_body(x_ref, v_ref, aggs_ref, aggv_ref, vmwT_ref, vmb_ref,
                 qwT_ref, qb_ref, kwT_ref, kb_ref, swT_ref, sb_ref,
                 vvwT_ref, vvb_ref,
                 x2_ref, v2_ref, q_ref, k_ref, vs_ref, p_ref):
    x2 = x_ref[...] + aggs_ref[...]
    v2 = (v_ref[...]
          + jnp.dot(aggv_ref[...], vmwT_ref[...],
                    preferred_element_type=jnp.float32) + vmb_ref[...])
    q, k, vs, p = _proj(x2, v2, qwT_ref[...], qb_ref[...], kwT_ref[...],
                        kb_ref[...], swT_ref[...], sb_ref[...],
                        vvwT_ref[...], vvb_ref[...])
    x2_ref[...], v2_ref[...] = x2, v2
    q_ref[...], k_ref[...], vs_ref[...], p_ref[...] = q, k, vs, p


def _tc_fin_body(x_ref, aggs_ref, x2_ref):
    x2_ref[...] = x_ref[...] + aggs_ref[...]


def _tc_init(nb, tok, coords, embed, vwT, vecb, lw):
    grid = nb // BLK
    nd = jax.ShapeDtypeStruct((nb, DIM), jnp.float32)
    return pl.pallas_call(
        _tc_init_body,
        grid=(grid,),
        in_specs=[_row_spec(BLK, 1), _row_spec(BLK, 3),
                  _full_spec((DIM, DIM)), _full_spec((3, DIM)),
                  _full_spec((1, DIM)),
                  _full_spec((DIM, DIM)), _full_spec((1, DIM)),
                  _full_spec((DIM, DIM)), _full_spec((1, DIM)),
                  _full_spec((DIM, DIM)), _full_spec((1, DIM)),
                  _full_spec((DIM, DIM)), _full_spec((1, DIM))],
        out_specs=[_row_spec(BLK, DIM)] * 6,
        out_shape=[nd] * 6,
        name="tc_init_proj",
    )(tok, coords, embed, vwT, vecb, *lw)


def _tc_mid(nb, x, v, aggs, aggv, vmwT, vmb, lw):
    grid = nb // BLK
    nd = jax.ShapeDtypeStruct((nb, DIM), jnp.float32)
    return pl.pallas_call(
        _tc_mid_body,
        grid=(grid,),
        in_specs=[_row_spec(BLK, DIM)] * 4
        + [_full_spec((DIM, DIM)), _full_spec((1, DIM)),
           _full_spec((DIM, DIM)), _full_spec((1, DIM)),
           _full_spec((DIM, DIM)), _full_spec((1, DIM)),
           _full_spec((DIM, DIM)), _full_spec((1, DIM)),
           _full_spec((DIM, DIM)), _full_spec((1, DIM))],
        out_specs=[_row_spec(BLK, DIM)] * 6,
        out_shape=[nd] * 6,
        name="tc_mid_proj",
    )(x, v, aggs, aggv, vmwT, vmb, *lw)


def _tc_fin(nb, x, aggs):
    grid = nb // BLK
    return pl.pallas_call(
        _tc_fin_body,
        grid=(grid,),
        in_specs=[_row_spec(BLK, DIM)] * 2,
        out_specs=_row_spec(BLK, DIM),
        out_shape=jax.ShapeDtypeStruct((nb, DIM), jnp.float32),
        name="tc_fin",
    )(x, aggs)


def kernel(src_tokens, padded_coordinates, src_distance, src_edge_type,
           edge_index, params):
    b, n = src_tokens.shape
    nb = b * n
    num_layers = len(params['layers'])

    tok = src_tokens.reshape(nb, 1).astype(jnp.int32)
    coords = padded_coordinates.reshape(nb, 3).astype(jnp.float32)
    rows = edge_index[0].astype(jnp.int32)
    cols = edge_index[1].astype(jnp.int32)
    e = rows.shape[0]
    epad = ((e + CHUNK + 7) // 8) * 8
    rows_p = jnp.concatenate(
        [rows, jnp.zeros((epad - e,), jnp.int32)])
    cols_p = jnp.concatenate(
        [cols, jnp.zeros((epad - e,), jnp.int32)])
    bounds = (jnp.arange(33, dtype=jnp.int32) * (nb // NTILES))
    off = jnp.searchsorted(rows, bounds, side='left').astype(jnp.int32)
    off48 = jnp.concatenate([off, jnp.full((15,), e, jnp.int32)])

    embed = params['embed']
    vwT = params['vec_w'].T
    vecb = params['vec_b'].reshape(1, DIM)

    def layer_weights(lp):
        kvw = lp['kv_w']
        kvb = lp['kv_b']
        return (lp['q_w'].T, lp['q_b'].reshape(1, DIM),
                kvw[0:DIM].T, kvb[0:DIM].reshape(1, DIM),
                kvw[DIM:2 * DIM].T, kvb[DIM:2 * DIM].reshape(1, DIM),
                kvw[2 * DIM:].T, kvb[2 * DIM:].reshape(1, DIM))

    lws = [layer_weights(lp) for lp in params['layers']]
    sc_edge = _make_sc_edge(nb, epad)

    x, v, q, k, vs, p = _tc_init(nb, tok, coords, embed, vwT, vecb, lws[0])
    for l in range(num_layers):
        aggs_f, aggv_f = sc_edge(q, k, vs, p, rows_p, cols_p, off48)
        aggs = aggs_f.reshape(nb, DIM)
        aggv = aggv_f.reshape(nb, DIM)
        if l + 1 < num_layers:
            lp = params['layers'][l]
            x, v, q, k, vs, p = _tc_mid(
                nb, x, v, aggs, aggv, lp['vm_w'].T,
                lp['vm_b'].reshape(1, DIM), lws[l + 1])
        else:
            x = _tc_fin(nb, x, aggs)

    encoder_rep = x.reshape(b, n, DIM)
    padding_mask = src_tokens == 0
    return encoder_rep, padding_mask


# speedup vs baseline: 2.1207x; 1.4966x over previous
import functools

import jax
import jax.numpy as jnp
from jax import lax
from jax.experimental import pallas as pl
from jax.experimental.pallas import tpu as pltpu
from jax.experimental.pallas import tpu_sc as plsc

DIM = 128
NSLICE = DIM // 16
CHUNK = 32
NTILES = 32
BLK = 512
PIB = "promise_in_bounds"


def _sc_edge_body(npt,
                  q_hbm, ksp_hbm, rows_hbm, cols_hbm, off_hbm,
                  aggs_hbm, aggv_hbm,
                  off_v, qtile, ksp0, ksp1, rows0, rows1, cols0, cols1,
                  accs, accv, semq, semg0, semg1, semi0, semi1):
    wid = lax.axis_index("s") * 2 + lax.axis_index("c")
    lane = lax.broadcasted_iota(jnp.int32, (16,), 0)
    base_node = wid * npt
    words = npt * DIM

    qcp = pltpu.async_copy(q_hbm.at[pl.ds(base_node, npt)], qtile, semq)
    pltpu.sync_copy(off_hbm, off_v)
    lo = off_v[pl.ds(wid, 16)][0]
    hi = off_v[pl.ds(wid + 1, 16)][0]
    start = jnp.bitwise_and(lo, jnp.int32(~7))
    nch = (hi - start + (CHUNK - 1)) // CHUNK
    npairs = (nch + 1) // 2

    zeros16 = jnp.zeros((16,), jnp.float32)

    def zbody(t, c):
        for u in range(4):
            accs[pl.ds(t * 64 + 16 * u, 16)] = zeros16
            accv[pl.ds(t * 64 + 16 * u, 16)] = zeros16
        return c

    lax.fori_loop(0, words // 64, zbody, 0)
    qcp.wait()

    def ds8(x):
        return pl.ds(pl.multiple_of(x, 8), CHUNK)

    def drain_ksp(sem, dst):
        pltpu.make_async_copy(ksp_hbm.at[pl.ds(0, CHUNK)], dst, sem).wait()

    def drain_idx(sem, dst):
        pltpu.make_async_copy(rows_hbm.at[pl.ds(0, CHUNK)], dst, sem).wait()

    def compute(e0c, r16a, r16b, kspb):
        for h in range(2):
            r16 = r16a if h == 0 else r16b
            for e in range(16):
                ei = 16 * h + e
                eid = e0c + ei
                validf = jnp.where(
                    jnp.logical_and(eid >= lo, eid < hi),
                    jnp.float32(1.0), jnp.float32(0.0))
                rloc = jnp.clip(r16[e] - base_node, 0, npt - 1)
                rb = rloc * DIM
                dot = qtile[rloc, pl.ds(0, 16)] * kspb[ei, pl.ds(0, 16)]
                for j in range(1, NSLICE):
                    dot = dot + (qtile[rloc, pl.ds(16 * j, 16)]
                                 * kspb[ei, pl.ds(16 * j, 16)])
                for sh in (8, 4, 2, 1):
                    dot = dot + dot.at[lane ^ sh].get(mode=PIB)
                w = dot * validf
                for j in range(NSLICE):
                    plsc.addupdate(
                        accs.at[pl.ds(rb + 16 * j, 16)],
                        w * kspb[ei, pl.ds(DIM + 16 * j, 16)])
                    plsc.addupdate(
                        accv.at[pl.ds(rb + 16 * j, 16)],
                        w * kspb[ei, pl.ds(2 * DIM + 16 * j, 16)])

    pltpu.sync_copy(rows_hbm.at[ds8(start)], rows0)
    pltpu.sync_copy(cols_hbm.at[ds8(start)], cols0)
    pltpu.async_copy(ksp_hbm.at[cols0], ksp0, semg0)
    pltpu.async_copy(rows_hbm.at[ds8(start + CHUNK)], rows1, semi1)
    pltpu.async_copy(cols_hbm.at[ds8(start + CHUNK)], cols1, semi1)

    def pbody(gg, c):
        ca = start + (2 * gg) * CHUNK
        drain_ksp(semg0, ksp0)
        ra0 = rows0[pl.ds(0, 16)]
        ra1 = rows0[pl.ds(16, 16)]
        drain_idx(semi1, rows1)
        drain_idx(semi1, cols1)
        pltpu.async_copy(ksp_hbm.at[cols1], ksp1, semg1)
        pltpu.async_copy(rows_hbm.at[ds8(ca + 2 * CHUNK)], rows0, semi0)
        pltpu.async_copy(cols_hbm.at[ds8(ca + 2 * CHUNK)], cols0, semi0)
        compute(ca, ra0, ra1, ksp0)
        drain_ksp(semg1, ksp1)
        rb0 = rows1[pl.ds(0, 16)]
        rb1 = rows1[pl.ds(16, 16)]
        drain_idx(semi0, rows0)
        drain_idx(semi0, cols0)
        pltpu.async_copy(ksp_hbm.at[cols0], ksp0, semg0)
        pltpu.async_copy(rows_hbm.at[ds8(ca + 3 * CHUNK)], rows1, semi1)
        pltpu.async_copy(cols_hbm.at[ds8(ca + 3 * CHUNK)], cols1, semi1)
        compute(ca + CHUNK, rb0, rb1, ksp1)
        return c

    lax.fori_loop(0, npairs, pbody, 0)

    drain_ksp(semg0, ksp0)
    drain_idx(semi1, rows1)
    drain_idx(semi1, cols1)

    pltpu.sync_copy(accs, aggs_hbm.at[pl.ds(wid * words, words)])
    pltpu.sync_copy(accv, aggv_hbm.at[pl.ds(wid * words, words)])


def _make_sc_edge(num_nodes):
    npt = num_nodes // NTILES
    words = npt * DIM
    mesh = plsc.VectorSubcoreMesh(core_axis_name="c", subcore_axis_name="s")
    return pl.kernel(
        functools.partial(_sc_edge_body, npt),
        out_type=[jax.ShapeDtypeStruct((num_nodes * DIM,), jnp.float32),
                  jax.ShapeDtypeStruct((num_nodes * DIM,), jnp.float32)],
        mesh=mesh,
        scratch_types=[
            pltpu.VMEM((48,), jnp.int32),
            pltpu.VMEM((npt, DIM), jnp.float32),
            pltpu.VMEM((CHUNK, 3 * DIM), jnp.float32),
            pltpu.VMEM((CHUNK, 3 * DIM), jnp.float32),
            pltpu.VMEM((CHUNK,), jnp.int32),
            pltpu.VMEM((CHUNK,), jnp.int32),
            pltpu.VMEM((CHUNK,), jnp.int32),
            pltpu.VMEM((CHUNK,), jnp.int32),
            pltpu.VMEM((words,), jnp.float32),
            pltpu.VMEM((words,), jnp.float32),
            pltpu.SemaphoreType.DMA,
            pltpu.SemaphoreType.DMA,
            pltpu.SemaphoreType.DMA,
            pltpu.SemaphoreType.DMA,
            pltpu.SemaphoreType.DMA,
        ],
        name="sc_edge_agg",
    )


def _row_spec(nrow, ncol):
    return pl.BlockSpec((nrow, ncol), lambda i: (i, 0))


def _full_spec(shape):
    return pl.BlockSpec(shape, lambda i: tuple(0 for _ in shape))


def _proj(x, v, qwT, qb, kwT, kb, swT, sb, vvwT, vvb, q_ref, ksp_ref):
    q_ref[...] = jnp.dot(x, qwT, preferred_element_type=jnp.float32) + qb
    ksp_ref[:, 0:DIM] = (
        jnp.dot(x, kwT, preferred_element_type=jnp.float32) + kb)
    ksp_ref[:, DIM:2 * DIM] = (
        jnp.dot(x, swT, preferred_element_type=jnp.float32) + sb)
    vv = jnp.dot(x, vvwT, preferred_element_type=jnp.float32) + vvb
    ksp_ref[:, 2 * DIM:3 * DIM] = vv * v


def _tc_init_body(tok_ref, coord_ref, embed_ref, vwT_ref, vecb_ref,
                  qwT_ref, qb_ref, kwT_ref, kb_ref, swT_ref, sb_ref,
                  vvwT_ref, vvb_ref,
                  x_ref, v_ref, q_ref, ksp_ref):
    tok = tok_ref[...]
    oh = (tok == lax.broadcasted_iota(jnp.int32, (1, DIM), 1)
          ).astype(jnp.float32)
    x = jnp.dot(oh, embed_ref[...], preferred_element_type=jnp.float32)
    c = coord_ref[...]
    vwT = vwT_ref[...]
    v = (vecb_ref[...] + c[:, 0:1] * vwT[0:1, :]
         + c[:, 1:2] * vwT[1:2, :] + c[:, 2:3] * vwT[2:3, :])
    x_ref[...], v_ref[...] = x, v
    _proj(x, v, qwT_ref[...], qb_ref[...], kwT_ref[...], kb_ref[...],
          swT_ref[...], sb_ref[...], vvwT_ref[...], vvb_ref[...],
          q_ref, ksp_ref)


def _tc_mid_body(x_ref, v_ref, aggs_ref, aggv_ref, vmwT_ref, vmb_ref,
                 qwT_ref, qb_ref, kwT_ref, kb_ref, swT_ref, sb_ref,
                 vvwT_ref, vvb_ref,
                 x2_ref, v2_ref, q_ref, ksp_ref):
    x2 = x_ref[...] + aggs_ref[...]
    v2 = (v_ref[...]
          + jnp.dot(aggv_ref[...], vmwT_ref[...],
                    preferred_element_type=jnp.float32) + vmb_ref[...])
    x2_ref[...], v2_ref[...] = x2, v2
    _proj(x2, v2, qwT_ref[...], qb_ref[...], kwT_ref[...], kb_ref[...],
          swT_ref[...], sb_ref[...], vvwT_ref[...], vvb_ref[...],
          q_ref, ksp_ref)


def _tc_fin_body(x_ref, aggs_ref, x2_ref):
    x2_ref[...] = x_ref[...] + aggs_ref[...]


def _tc_init(nb, tok, coords, embed, vwT, vecb, lw):
    grid = nb // BLK
    nd = jax.ShapeDtypeStruct((nb, DIM), jnp.float32)
    ksp = jax.ShapeDtypeStruct((nb, 3 * DIM), jnp.float32)
    return pl.pallas_call(
        _tc_init_body,
        grid=(grid,),
        in_specs=[_row_spec(BLK, 1), _row_spec(BLK, 3),
                  _full_spec((DIM, DIM)), _full_spec((3, DIM)),
                  _full_spec((1, DIM)),
                  _full_spec((DIM, DIM)), _full_spec((1, DIM)),
                  _full_spec((DIM, DIM)), _full_spec((1, DIM)),
                  _full_spec((DIM, DIM)), _full_spec((1, DIM)),
                  _full_spec((DIM, DIM)), _full_spec((1, DIM))],
        out_specs=[_row_spec(BLK, DIM)] * 3 + [_row_spec(BLK, 3 * DIM)],
        out_shape=[nd, nd, nd, ksp],
        name="tc_init_proj",
    )(tok, coords, embed, vwT, vecb, *lw)


def _tc_mid(nb, x, v, aggs, aggv, vmwT, vmb, lw):
    grid = nb // BLK
    nd = jax.ShapeDtypeStruct((nb, DIM), jnp.float32)
    ksp = jax.ShapeDtypeStruct((nb, 3 * DIM), jnp.float32)
    return pl.pallas_call(
        _tc_mid_body,
        grid=(grid,),
        in_specs=[_row_spec(BLK, DIM)] * 4
        + [_full_spec((DIM, DIM)), _full_spec((1, DIM)),
           _full_spec((DIM, DIM)), _full_spec((1, DIM)),
           _full_spec((DIM, DIM)), _full_spec((1, DIM)),
           _full_spec((DIM, DIM)), _full_spec((1, DIM)),
           _full_spec((DIM, DIM)), _full_spec((1, DIM))],
        out_specs=[_row_spec(BLK, DIM)] * 3 + [_row_spec(BLK, 3 * DIM)],
        out_shape=[nd, nd, nd, ksp],
        name="tc_mid_proj",
    )(x, v, aggs, aggv, vmwT, vmb, *lw)


def _tc_fin(nb, x, aggs):
    grid = nb // BLK
    return pl.pallas_call(
        _tc_fin_body,
        grid=(grid,),
        in_specs=[_row_spec(BLK, DIM)] * 2,
        out_specs=_row_spec(BLK, DIM),
        out_shape=jax.ShapeDtypeStruct((nb, DIM), jnp.float32),
        name="tc_fin",
    )(x, aggs)


def kernel(src_tokens, padded_coordinates, src_distance, src_edge_type,
           edge_index, params):
    b, n = src_tokens.shape
    nb = b * n
    num_layers = len(params['layers'])

    tok = src_tokens.reshape(nb, 1).astype(jnp.int32)
    coords = padded_coordinates.reshape(nb, 3).astype(jnp.float32)
    rows = edge_index[0].astype(jnp.int32)
    cols = edge_index[1].astype(jnp.int32)
    e = rows.shape[0]
    epad = ((e + 4 * CHUNK + 7) // 8) * 8 + 8
    rows_p = jnp.concatenate([rows, jnp.zeros((epad - e,), jnp.int32)])
    cols_p = jnp.concatenate([cols, jnp.zeros((epad - e,), jnp.int32)])
    bounds = (jnp.arange(NTILES + 1, dtype=jnp.int32) * (nb // NTILES))
    off = jnp.searchsorted(rows, bounds, side='left').astype(jnp.int32)
    off48 = jnp.concatenate([off, jnp.full((15,), e, jnp.int32)])

    embed = params['embed']
    vwT = params['vec_w'].T
    vecb = params['vec_b'].reshape(1, DIM)

    def layer_weights(lp):
        kvw = lp['kv_w']
        kvb = lp['kv_b']
        return (lp['q_w'].T, lp['q_b'].reshape(1, DIM),
                kvw[0:DIM].T, kvb[0:DIM].reshape(1, DIM),
                kvw[DIM:2 * DIM].T, kvb[DIM:2 * DIM].reshape(1, DIM),
                kvw[2 * DIM:].T, kvb[2 * DIM:].reshape(1, DIM))

    lws = [layer_weights(lp) for lp in params['layers']]
    sc_edge = _make_sc_edge(nb)

    x, v, q, ksp = _tc_init(nb, tok, coords, embed, vwT, vecb, lws[0])
    for l in range(num_layers):
        aggs_f, aggv_f = sc_edge(q, ksp, rows_p, cols_p, off48)
        aggs = aggs_f.reshape(nb, DIM)
        aggv = aggv_f.reshape(nb, DIM)
        if l + 1 < num_layers:
            lp = params['layers'][l]
            x, v, q, ksp = _tc_mid(
                nb, x, v, aggs, aggv, lp['vm_w'].T,
                lp['vm_b'].reshape(1, DIM), lws[l + 1])
        else:
            x = _tc_fin(nb, x, aggs)

    encoder_rep = x.reshape(b, n, DIM)
    padding_mask = src_tokens == 0
    return encoder_rep, padding_mask


# speedup vs baseline: 2.6958x; 1.2711x over previous
import functools

import jax
import jax.numpy as jnp
from jax import lax
from jax.experimental import pallas as pl
from jax.experimental.pallas import tpu as pltpu
from jax.experimental.pallas import tpu_sc as plsc

DIM = 128
NSLICE = DIM // 16
CHUNK = 32
NTILES = 32
BLK = 512


def _sc_edge_body(npt, nlog,
                  w_hbm, sp_hbm, rows_hbm, cols_hbm, off_hbm,
                  aggs_hbm, aggv_hbm,
                  off_v, sp0, sp1, w0, w1, wi0, wi1,
                  rows0, rows1, cols0, cols1,
                  accs, accv, semg0, semg1, semi0, semi1):
    wid = lax.axis_index("s") * 2 + lax.axis_index("c")
    base_node = wid * npt
    words = npt * DIM
    nmask = (1 << nlog) - 1

    pltpu.sync_copy(off_hbm, off_v)
    lo = off_v[pl.ds(wid, 16)][0]
    hi = off_v[pl.ds(wid + 1, 16)][0]
    start = jnp.bitwise_and(lo, jnp.int32(~7))
    nch = (hi - start + (CHUNK - 1)) // CHUNK
    npairs = (nch + 1) // 2

    zeros16 = jnp.zeros((16,), jnp.float32)

    def zbody(t, c):
        for u in range(4):
            accs[pl.ds(t * 64 + 16 * u, 16)] = zeros16
            accv[pl.ds(t * 64 + 16 * u, 16)] = zeros16
        return c

    lax.fori_loop(0, words // 64, zbody, 0)

    def ds8(x):
        return pl.ds(pl.multiple_of(x, 8), CHUNK)

    def drain(dst, sem):
        pltpu.make_async_copy(rows_hbm.at[pl.ds(0, CHUNK)], dst, sem).wait()

    def drain_sp(dst, sem):
        pltpu.make_async_copy(sp_hbm.at[pl.ds(0, CHUNK)], dst, sem).wait()

    def drain_w(dst, sem):
        pltpu.make_async_copy(w_hbm.at[pl.ds(0, CHUNK)], dst, sem).wait()

    def fire_gathers(rowsr, colsr, wir, wr, spr, sem):
        for h in range(CHUNK // 16):
            r16 = rowsr[pl.ds(16 * h, 16)]
            c16 = colsr[pl.ds(16 * h, 16)]
            widx = ((jnp.bitwise_and(r16, nmask) << nlog)
                    + jnp.bitwise_and(c16, nmask)
                    + ((r16 >> nlog) << (2 * nlog)))
            wir[pl.ds(16 * h, 16)] = widx
        pltpu.async_copy(w_hbm.at[wir], wr, sem)
        pltpu.async_copy(sp_hbm.at[colsr], spr, sem)

    def compute(e0c, r16a, r16b, wr, spr):
        wall = [wr[pl.ds(0, 16)], wr[pl.ds(16, 16)]]
        for h in range(2):
            r16 = r16a if h == 0 else r16b
            w16 = wall[h]
            for e in range(16):
                ei = 16 * h + e
                eid = e0c + ei
                validf = jnp.where(
                    jnp.logical_and(eid >= lo, eid < hi),
                    jnp.float32(1.0), jnp.float32(0.0))
                w = w16[e] * validf
                rb = jnp.clip(r16[e] - base_node, 0, npt - 1) * DIM
                for j in range(NSLICE):
                    plsc.addupdate(
                        accs.at[pl.ds(rb + 16 * j, 16)],
                        w * spr[ei, pl.ds(16 * j, 16)])
                    plsc.addupdate(
                        accv.at[pl.ds(rb + 16 * j, 16)],
                        w * spr[ei, pl.ds(DIM + 16 * j, 16)])

    pltpu.sync_copy(rows_hbm.at[ds8(start)], rows0)
    pltpu.sync_copy(cols_hbm.at[ds8(start)], cols0)
    fire_gathers(rows0, cols0, wi0, w0, sp0, semg0)
    pltpu.async_copy(rows_hbm.at[ds8(start + CHUNK)], rows1, semi1)
    pltpu.async_copy(cols_hbm.at[ds8(start + CHUNK)], cols1, semi1)

    def pbody(gg, c):
        ca = start + (2 * gg) * CHUNK
        drain_w(w0, semg0)
        drain_sp(sp0, semg0)
        ra0 = rows0[pl.ds(0, 16)]
        ra1 = rows0[pl.ds(16, 16)]
        drain(rows1, semi1)
        drain(cols1, semi1)
        fire_gathers(rows1, cols1, wi1, w1, sp1, semg1)
        pltpu.async_copy(rows_hbm.at[ds8(ca + 2 * CHUNK)], rows0, semi0)
        pltpu.async_copy(cols_hbm.at[ds8(ca + 2 * CHUNK)], cols0, semi0)
        compute(ca, ra0, ra1, w0, sp0)
        drain_w(w1, semg1)
        drain_sp(sp1, semg1)
        rb0 = rows1[pl.ds(0, 16)]
        rb1 = rows1[pl.ds(16, 16)]
        drain(rows0, semi0)
        drain(cols0, semi0)
        fire_gathers(rows0, cols0, wi0, w0, sp0, semg0)
        pltpu.async_copy(rows_hbm.at[ds8(ca + 3 * CHUNK)], rows1, semi1)
        pltpu.async_copy(cols_hbm.at[ds8(ca + 3 * CHUNK)], cols1, semi1)
        compute(ca + CHUNK, rb0, rb1, w1, sp1)
        return c

    lax.fori_loop(0, npairs, pbody, 0)

    drain_w(w0, semg0)
    drain_sp(sp0, semg0)
    drain(rows1, semi1)
    drain(cols1, semi1)

    pltpu.sync_copy(accs, aggs_hbm.at[pl.ds(wid * words, words)])
    pltpu.sync_copy(accv, aggv_hbm.at[pl.ds(wid * words, words)])


def _make_sc_edge(num_nodes, n_per_batch):
    npt = num_nodes // NTILES
    words = npt * DIM
    nlog = n_per_batch.bit_length() - 1
    assert (1 << nlog) == n_per_batch
    mesh = plsc.VectorSubcoreMesh(core_axis_name="c", subcore_axis_name="s")
    return pl.kernel(
        functools.partial(_sc_edge_body, npt, nlog),
        out_type=[jax.ShapeDtypeStruct((num_nodes * DIM,), jnp.float32),
                  jax.ShapeDtypeStruct((num_nodes * DIM,), jnp.float32)],
        mesh=mesh,
        scratch_types=[
            pltpu.VMEM((48,), jnp.int32),
            pltpu.VMEM((CHUNK, 2 * DIM), jnp.float32),
            pltpu.VMEM((CHUNK, 2 * DIM), jnp.float32),
            pltpu.VMEM((CHUNK,), jnp.float32),
            pltpu.VMEM((CHUNK,), jnp.float32),
            pltpu.VMEM((CHUNK,), jnp.int32),
            pltpu.VMEM((CHUNK,), jnp.int32),
            pltpu.VMEM((CHUNK,), jnp.int32),
            pltpu.VMEM((CHUNK,), jnp.int32),
            pltpu.VMEM((CHUNK,), jnp.int32),
            pltpu.VMEM((CHUNK,), jnp.int32),
            pltpu.VMEM((words,), jnp.float32),
            pltpu.VMEM((words,), jnp.float32),
            pltpu.SemaphoreType.DMA,
            pltpu.SemaphoreType.DMA,
            pltpu.SemaphoreType.DMA,
            pltpu.SemaphoreType.DMA,
        ],
        name="sc_edge_agg",
    )


def _row_spec(nrow, ncol):
    return pl.BlockSpec((nrow, ncol), lambda i: (i, 0))


def _full_spec(shape):
    return pl.BlockSpec(shape, lambda i: tuple(0 for _ in shape))


def _proj(x, v, qwT, qb, kwT, kb, swT, sb, vvwT, vvb, q_ref, k_ref, sp_ref):
    q_ref[...] = jnp.dot(x, qwT, preferred_element_type=jnp.float32) + qb
    k_ref[...] = jnp.dot(x, kwT, preferred_element_type=jnp.float32) + kb
    sp_ref[:, 0:DIM] = (
        jnp.dot(x, swT, preferred_element_type=jnp.float32) + sb)
    vv = jnp.dot(x, vvwT, preferred_element_type=jnp.float32) + vvb
    sp_ref[:, DIM:2 * DIM] = vv * v


def _tc_init_body(tok_ref, coord_ref, embed_ref, vwT_ref, vecb_ref,
                  qwT_ref, qb_ref, kwT_ref, kb_ref, swT_ref, sb_ref,
                  vvwT_ref, vvb_ref,
                  x_ref, v_ref, q_ref, k_ref, sp_ref):
    tok = tok_ref[...]
    oh = (tok == lax.broadcasted_iota(jnp.int32, (1, DIM), 1)
          ).astype(jnp.float32)
    x = jnp.dot(oh, embed_ref[...], preferred_element_type=jnp.float32)
    c = coord_ref[...]
    vwT = vwT_ref[...]
    v = (vecb_ref[...] + c[:, 0:1] * vwT[0:1, :]
         + c[:, 1:2] * vwT[1:2, :] + c[:, 2:3] * vwT[2:3, :])
    x_ref[...], v_ref[...] = x, v
    _proj(x, v, qwT_ref[...], qb_ref[...], kwT_ref[...], kb_ref[...],
          swT_ref[...], sb_ref[...], vvwT_ref[...], vvb_ref[...],
          q_ref, k_ref, sp_ref)


def _tc_mid_body(x_ref, v_ref, aggs_ref, aggv_ref, vmwT_ref, vmb_ref,
                 qwT_ref, qb_ref, kwT_ref, kb_ref, swT_ref, sb_ref,
                 vvwT_ref, vvb_ref,
                 x2_ref, v2_ref, q_ref, k_ref, sp_ref):
    x2 = x_ref[...] + aggs_ref[...]
    v2 = (v_ref[...]
          + jnp.dot(aggv_ref[...], vmwT_ref[...],
                    preferred_element_type=jnp.float32) + vmb_ref[...])
    x2_ref[...], v2_ref[...] = x2, v2
    _proj(x2, v2, qwT_ref[...], qb_ref[...], kwT_ref[...], kb_ref[...],
          swT_ref[...], sb_ref[...], vvwT_ref[...], vvb_ref[...],
          q_ref, k_ref, sp_ref)


def _tc_fin_body(x_ref, aggs_ref, x2_ref):
    x2_ref[...] = x_ref[...] + aggs_ref[...]


def _tc_w_body(q_ref, k_ref, w_ref):
    w_ref[...] = lax.dot_general(
        q_ref[...], k_ref[...], (((1,), (1,)), ((), ())),
        preferred_element_type=jnp.float32)


def _tc_init(nb, tok, coords, embed, vwT, vecb, lw):
    grid = nb // BLK
    nd = jax.ShapeDtypeStruct((nb, DIM), jnp.float32)
    sp = jax.ShapeDtypeStruct((nb, 2 * DIM), jnp.float32)
    return pl.pallas_call(
        _tc_init_body,
        grid=(grid,),
        in_specs=[_row_spec(BLK, 1), _row_spec(BLK, 3),
                  _full_spec((DIM, DIM)), _full_spec((3, DIM)),
                  _full_spec((1, DIM)),
                  _full_spec((DIM, DIM)), _full_spec((1, DIM)),
                  _full_spec((DIM, DIM)), _full_spec((1, DIM)),
                  _full_spec((DIM, DIM)), _full_spec((1, DIM)),
                  _full_spec((DIM, DIM)), _full_spec((1, DIM))],
        out_specs=[_row_spec(BLK, DIM)] * 4 + [_row_spec(BLK, 2 * DIM)],
        out_shape=[nd, nd, nd, nd, sp],
        name="tc_init_proj",
    )(tok, coords, embed, vwT, vecb, *lw)


def _tc_mid(nb, x, v, aggs, aggv, vmwT, vmb, lw):
    grid = nb // BLK
    nd = jax.ShapeDtypeStruct((nb, DIM), jnp.float32)
    sp = jax.ShapeDtypeStruct((nb, 2 * DIM), jnp.float32)
    return pl.pallas_call(
        _tc_mid_body,
        grid=(grid,),
        in_specs=[_row_spec(BLK, DIM)] * 4
        + [_full_spec((DIM, DIM)), _full_spec((1, DIM)),
           _full_spec((DIM, DIM)), _full_spec((1, DIM)),
           _full_spec((DIM, DIM)), _full_spec((1, DIM)),
           _full_spec((DIM, DIM)), _full_spec((1, DIM)),
           _full_spec((DIM, DIM)), _full_spec((1, DIM))],
        out_specs=[_row_spec(BLK, DIM)] * 4 + [_row_spec(BLK, 2 * DIM)],
        out_shape=[nd, nd, nd, nd, sp],
        name="tc_mid_proj",
    )(x, v, aggs, aggv, vmwT, vmb, *lw)


def _tc_w(nb, n, q, k):
    nbat = nb // n
    grid_r = n // BLK
    return pl.pallas_call(
        _tc_w_body,
        grid=(nbat, grid_r),
        in_specs=[pl.BlockSpec((BLK, DIM), lambda b, r: (b * grid_r + r, 0)),
                  pl.BlockSpec((n, DIM), lambda b, r: (b, 0))],
        out_specs=pl.BlockSpec((BLK, n), lambda b, r: (b * grid_r + r, 0)),
        out_shape=jax.ShapeDtypeStruct((nb, n), jnp.float32),
        name="tc_qkT",
    )(q, k)


def _tc_fin(nb, x, aggs):
    grid = nb // BLK
    return pl.pallas_call(
        _tc_fin_body,
        grid=(grid,),
        in_specs=[_row_spec(BLK, DIM)] * 2,
        out_specs=_row_spec(BLK, DIM),
        out_shape=jax.ShapeDtypeStruct((nb, DIM), jnp.float32),
        name="tc_fin",
    )(x, aggs)


def kernel(src_tokens, padded_coordinates, src_distance, src_edge_type,
           edge_index, params):
    b, n = src_tokens.shape
    nb = b * n
    num_layers = len(params['layers'])

    tok = src_tokens.reshape(nb, 1).astype(jnp.int32)
    coords = padded_coordinates.reshape(nb, 3).astype(jnp.float32)
    rows = edge_index[0].astype(jnp.int32)
    cols = edge_index[1].astype(jnp.int32)
    e = rows.shape[0]
    epad = ((e + 4 * CHUNK + 7) // 8) * 8 + 8
    rows_p = jnp.concatenate([rows, jnp.zeros((epad - e,), jnp.int32)])
    cols_p = jnp.concatenate([cols, jnp.zeros((epad - e,), jnp.int32)])
    bounds = (jnp.arange(NTILES + 1, dtype=jnp.int32) * (nb // NTILES))
    off = jnp.searchsorted(rows, bounds, side='left').astype(jnp.int32)
    off48 = jnp.concatenate([off, jnp.full((15,), e, jnp.int32)])

    embed = params['embed']
    vwT = params['vec_w'].T
    vecb = params['vec_b'].reshape(1, DIM)

    def layer_weights(lp):
        kvw = lp['kv_w']
        kvb = lp['kv_b']
        return (lp['q_w'].T, lp['q_b'].reshape(1, DIM),
                kvw[0:DIM].T, kvb[0:DIM].reshape(1, DIM),
                kvw[DIM:2 * DIM].T, kvb[DIM:2 * DIM].reshape(1, DIM),
                kvw[2 * DIM:].T, kvb[2 * DIM:].reshape(1, DIM))

    lws = [layer_weights(lp) for lp in params['layers']]
    sc_edge = _make_sc_edge(nb, n)

    x, v, q, k, sp = _tc_init(nb, tok, coords, embed, vwT, vecb, lws[0])
    for l in range(num_layers):
        w = _tc_w(nb, n, q, k)
        aggs_f, aggv_f = sc_edge(w.reshape(nb * n), sp, rows_p, cols_p,
                                 off48)
        aggs = aggs_f.reshape(nb, DIM)
        aggv = aggv_f.reshape(nb, DIM)
        if l + 1 < num_layers:
            lp = params['layers'][l]
            x, v, q, k, sp = _tc_mid(
                nb, x, v, aggs, aggv, lp['vm_w'].T,
                lp['vm_b'].reshape(1, DIM), lws[l + 1])
        else:
            x = _tc_fin(nb, x, aggs)

    encoder_rep = x.reshape(b, n, DIM)
    padding_mask = src_tokens == 0
    return encoder_rep, padding_mask


# speedup vs baseline: 5.7796x; 2.1440x over previous
import functools

import jax
import jax.numpy as jnp
from jax import lax
from jax.experimental import pallas as pl
from jax.experimental.pallas import tpu as pltpu
from jax.experimental.pallas import tpu_sc as plsc

DIM = 128
NSLICE = DIM // 16
CHUNK = 32
NTILES = 32
BLK = 512


def _sc_edge_body(npt, nlog,
                  w_hbm, sp_hbm, rows_hbm, cols_hbm, off_hbm,
                  aggs_hbm, aggv_hbm,
                  off_v, sp0, sp1, w0, w1, wi0, wi1,
                  rows0, rows1, cols0, cols1,
                  accs, accv, semg0, semg1, semi0, semi1):
    wid = lax.axis_index("s") * 2 + lax.axis_index("c")
    base_node = wid * npt
    words = npt * DIM
    nmask = (1 << nlog) - 1

    pltpu.sync_copy(off_hbm, off_v)
    lo = off_v[pl.ds(wid, 16)][0]
    hi = off_v[pl.ds(wid + 1, 16)][0]
    start = jnp.bitwise_and(lo, jnp.int32(~7))
    nch = (hi - start + (CHUNK - 1)) // CHUNK
    npairs = (nch + 1) // 2

    zeros16 = jnp.zeros((16,), jnp.float32)

    def zbody(t, c):
        for u in range(4):
            accs[pl.ds(t * 64 + 16 * u, 16)] = zeros16
            accv[pl.ds(t * 64 + 16 * u, 16)] = zeros16
        return c

    lax.fori_loop(0, words // 64, zbody, 0)

    def ds8(x):
        return pl.ds(pl.multiple_of(x, 8), CHUNK)

    def drain(dst, sem):
        pltpu.make_async_copy(rows_hbm.at[pl.ds(0, CHUNK)], dst, sem).wait()

    def drain_sp(dst, sem):
        pltpu.make_async_copy(sp_hbm.at[pl.ds(0, CHUNK)], dst, sem).wait()

    def drain_w(dst, sem):
        pltpu.make_async_copy(w_hbm.at[pl.ds(0, CHUNK)], dst, sem).wait()

    def fire_gathers(rowsr, colsr, wir, wr, spr, sem):
        for h in range(CHUNK // 16):
            r16 = rowsr[pl.ds(16 * h, 16)]
            c16 = colsr[pl.ds(16 * h, 16)]
            widx = ((jnp.bitwise_and(r16, nmask) << nlog)
                    + jnp.bitwise_and(c16, nmask)
                    + ((r16 >> nlog) << (2 * nlog)))
            wir[pl.ds(16 * h, 16)] = widx
        pltpu.async_copy(w_hbm.at[wir], wr, sem)
        pltpu.async_copy(sp_hbm.at[colsr], spr, sem)

    lane = lax.broadcasted_iota(jnp.int32, (16,), 0)

    def compute(e0c, rvecs, wr, spr, carry):
        prev_rb, cs, cp = carry
        for h in range(CHUNK // 16):
            r16 = rvecs[h]
            w16 = wr[pl.ds(16 * h, 16)]
            eid16 = (e0c + 16 * h) + lane
            validm = jnp.where(
                jnp.logical_and(eid16 >= lo, eid16 < hi),
                jnp.float32(1.0), jnp.float32(0.0))
            wm = w16 * validm
            rloc = jnp.clip(r16 - base_node, 0, npt - 1) * DIM
            for e in range(16):
                ei = 16 * h + e
                rb_e = rloc[e]
                changed = rb_e != prev_rb

                @pl.when(changed)
                def _flush(prev_rb=prev_rb, cs=cs, cp=cp):
                    for j in range(NSLICE):
                        accs[pl.ds(prev_rb + 16 * j, 16)] = cs[j]
                        accv[pl.ds(prev_rb + 16 * j, 16)] = cp[j]

                keep = jnp.where(changed, jnp.float32(0.0),
                                 jnp.float32(1.0))
                sel = jnp.full((16,), e, jnp.int32)
                w_b = wm.at[sel].get(mode="promise_in_bounds")
                cs = tuple(
                    cs[j] * keep + w_b * spr[ei, pl.ds(16 * j, 16)]
                    for j in range(NSLICE))
                cp = tuple(
                    cp[j] * keep + w_b * spr[ei, pl.ds(DIM + 16 * j, 16)]
                    for j in range(NSLICE))
                prev_rb = rb_e
        return prev_rb, cs, cp

    pltpu.sync_copy(rows_hbm.at[ds8(start)], rows0)
    pltpu.sync_copy(cols_hbm.at[ds8(start)], cols0)
    fire_gathers(rows0, cols0, wi0, w0, sp0, semg0)
    pltpu.async_copy(rows_hbm.at[ds8(start + CHUNK)], rows1, semi1)
    pltpu.async_copy(cols_hbm.at[ds8(start + CHUNK)], cols1, semi1)

    def pbody(gg, carry):
        ca = start + (2 * gg) * CHUNK
        drain_w(w0, semg0)
        drain_sp(sp0, semg0)
        ra = [rows0[pl.ds(16 * h, 16)] for h in range(CHUNK // 16)]
        drain(rows1, semi1)
        drain(cols1, semi1)
        fire_gathers(rows1, cols1, wi1, w1, sp1, semg1)
        pltpu.async_copy(rows_hbm.at[ds8(ca + 2 * CHUNK)], rows0, semi0)
        pltpu.async_copy(cols_hbm.at[ds8(ca + 2 * CHUNK)], cols0, semi0)
        carry = compute(ca, ra, w0, sp0, carry)
        drain_w(w1, semg1)
        drain_sp(sp1, semg1)
        rb = [rows1[pl.ds(16 * h, 16)] for h in range(CHUNK // 16)]
        drain(rows0, semi0)
        drain(cols0, semi0)
        fire_gathers(rows0, cols0, wi0, w0, sp0, semg0)
        pltpu.async_copy(rows_hbm.at[ds8(ca + 3 * CHUNK)], rows1, semi1)
        pltpu.async_copy(cols_hbm.at[ds8(ca + 3 * CHUNK)], cols1, semi1)
        carry = compute(ca + CHUNK, rb, w1, sp1, carry)
        return carry

    zero8 = tuple(jnp.zeros((16,), jnp.float32) for _ in range(NSLICE))
    carry0 = (jnp.int32(0), zero8, zero8)
    prev_rb, cs, cp = lax.fori_loop(0, npairs, pbody, carry0)

    for j in range(NSLICE):
        accs[pl.ds(prev_rb + 16 * j, 16)] = cs[j]
        accv[pl.ds(prev_rb + 16 * j, 16)] = cp[j]

    drain_w(w0, semg0)
    drain_sp(sp0, semg0)
    drain(rows1, semi1)
    drain(cols1, semi1)

    pltpu.sync_copy(accs, aggs_hbm.at[pl.ds(wid * words, words)])
    pltpu.sync_copy(accv, aggv_hbm.at[pl.ds(wid * words, words)])


def _make_sc_edge(num_nodes, n_per_batch):
    npt = num_nodes // NTILES
    words = npt * DIM
    nlog = n_per_batch.bit_length() - 1
    assert (1 << nlog) == n_per_batch
    mesh = plsc.VectorSubcoreMesh(core_axis_name="c", subcore_axis_name="s")
    return pl.kernel(
        functools.partial(_sc_edge_body, npt, nlog),
        out_type=[jax.ShapeDtypeStruct((num_nodes * DIM,), jnp.float32),
                  jax.ShapeDtypeStruct((num_nodes * DIM,), jnp.float32)],
        mesh=mesh,
        scratch_types=[
            pltpu.VMEM((48,), jnp.int32),
            pltpu.VMEM((CHUNK, 2 * DIM), jnp.float32),
            pltpu.VMEM((CHUNK, 2 * DIM), jnp.float32),
            pltpu.VMEM((CHUNK,), jnp.float32),
            pltpu.VMEM((CHUNK,), jnp.float32),
            pltpu.VMEM((CHUNK,), jnp.int32),
            pltpu.VMEM((CHUNK,), jnp.int32),
            pltpu.VMEM((CHUNK,), jnp.int32),
            pltpu.VMEM((CHUNK,), jnp.int32),
            pltpu.VMEM((CHUNK,), jnp.int32),
            pltpu.VMEM((CHUNK,), jnp.int32),
            pltpu.VMEM((words,), jnp.float32),
            pltpu.VMEM((words,), jnp.float32),
            pltpu.SemaphoreType.DMA,
            pltpu.SemaphoreType.DMA,
            pltpu.SemaphoreType.DMA,
            pltpu.SemaphoreType.DMA,
        ],
        name="sc_edge_agg",
    )


def _row_spec(nrow, ncol):
    return pl.BlockSpec((nrow, ncol), lambda i: (i, 0))


def _full_spec(shape):
    return pl.BlockSpec(shape, lambda i: tuple(0 for _ in shape))


def _proj(x, v, qwT, qb, kwT, kb, swT, sb, vvwT, vvb, q_ref, k_ref, sp_ref):
    q_ref[...] = jnp.dot(x, qwT, preferred_element_type=jnp.float32) + qb
    k_ref[...] = jnp.dot(x, kwT, preferred_element_type=jnp.float32) + kb
    sp_ref[:, 0:DIM] = (
        jnp.dot(x, swT, preferred_element_type=jnp.float32) + sb)
    vv = jnp.dot(x, vvwT, preferred_element_type=jnp.float32) + vvb
    sp_ref[:, DIM:2 * DIM] = vv * v


def _tc_init_body(tok_ref, coord_ref, embed_ref, vwT_ref, vecb_ref,
                  qwT_ref, qb_ref, kwT_ref, kb_ref, swT_ref, sb_ref,
                  vvwT_ref, vvb_ref,
                  x_ref, v_ref, q_ref, k_ref, sp_ref):
    tok = tok_ref[...]
    oh = (tok == lax.broadcasted_iota(jnp.int32, (1, DIM), 1)
          ).astype(jnp.float32)
    x = jnp.dot(oh, embed_ref[...], preferred_element_type=jnp.float32)
    c = coord_ref[...]
    vwT = vwT_ref[...]
    v = (vecb_ref[...] + c[:, 0:1] * vwT[0:1, :]
         + c[:, 1:2] * vwT[1:2, :] + c[:, 2:3] * vwT[2:3, :])
    x_ref[...], v_ref[...] = x, v
    _proj(x, v, qwT_ref[...], qb_ref[...], kwT_ref[...], kb_ref[...],
          swT_ref[...], sb_ref[...], vvwT_ref[...], vvb_ref[...],
          q_ref, k_ref, sp_ref)


def _tc_mid_body(x_ref, v_ref, aggs_ref, aggv_ref, vmwT_ref, vmb_ref,
                 qwT_ref, qb_ref, kwT_ref, kb_ref, swT_ref, sb_ref,
                 vvwT_ref, vvb_ref,
                 x2_ref, v2_ref, q_ref, k_ref, sp_ref):
    x2 = x_ref[...] + aggs_ref[...]
    v2 = (v_ref[...]
          + jnp.dot(aggv_ref[...], vmwT_ref[...],
                    preferred_element_type=jnp.float32) + vmb_ref[...])
    x2_ref[...], v2_ref[...] = x2, v2
    _proj(x2, v2, qwT_ref[...], qb_ref[...], kwT_ref[...], kb_ref[...],
          swT_ref[...], sb_ref[...], vvwT_ref[...], vvb_ref[...],
          q_ref, k_ref, sp_ref)


def _tc_fin_body(x_ref, aggs_ref, x2_ref):
    x2_ref[...] = x_ref[...] + aggs_ref[...]


def _tc_w_body(q_ref, k_ref, w_ref):
    res = lax.dot_general(
        q_ref[...], k_ref[...], (((1,), (1,)), ((), ())),
        preferred_element_type=jnp.float32)
    w_ref[...] = res.reshape(w_ref.shape)


def _tc_init(nb, tok, coords, embed, vwT, vecb, lw):
    grid = nb // BLK
    nd = jax.ShapeDtypeStruct((nb, DIM), jnp.float32)
    sp = jax.ShapeDtypeStruct((nb, 2 * DIM), jnp.float32)
    return pl.pallas_call(
        _tc_init_body,
        grid=(grid,),
        in_specs=[_row_spec(BLK, 1), _row_spec(BLK, 3),
                  _full_spec((DIM, DIM)), _full_spec((3, DIM)),
                  _full_spec((1, DIM)),
                  _full_spec((DIM, DIM)), _full_spec((1, DIM)),
                  _full_spec((DIM, DIM)), _full_spec((1, DIM)),
                  _full_spec((DIM, DIM)), _full_spec((1, DIM)),
                  _full_spec((DIM, DIM)), _full_spec((1, DIM))],
        out_specs=[_row_spec(BLK, DIM)] * 4 + [_row_spec(BLK, 2 * DIM)],
        out_shape=[nd, nd, nd, nd, sp],
        name="tc_init_proj",
    )(tok, coords, embed, vwT, vecb, *lw)


def _tc_mid(nb, x, v, aggs, aggv, vmwT, vmb, lw):
    grid = nb // BLK
    nd = jax.ShapeDtypeStruct((nb, DIM), jnp.float32)
    sp = jax.ShapeDtypeStruct((nb, 2 * DIM), jnp.float32)
    return pl.pallas_call(
        _tc_mid_body,
        grid=(grid,),
        in_specs=[_row_spec(BLK, DIM)] * 4
        + [_full_spec((DIM, DIM)), _full_spec((1, DIM)),
           _full_spec((DIM, DIM)), _full_spec((1, DIM)),
           _full_spec((DIM, DIM)), _full_spec((1, DIM)),
           _full_spec((DIM, DIM)), _full_spec((1, DIM)),
           _full_spec((DIM, DIM)), _full_spec((1, DIM))],
        out_specs=[_row_spec(BLK, DIM)] * 4 + [_row_spec(BLK, 2 * DIM)],
        out_shape=[nd, nd, nd, nd, sp],
        name="tc_mid_proj",
    )(x, v, aggs, aggv, vmwT, vmb, *lw)


def _tc_w(nb, n, q, k):
    nbat = nb // n
    grid_r = n // BLK
    return pl.pallas_call(
        _tc_w_body,
        grid=(nbat, grid_r),
        in_specs=[pl.BlockSpec((BLK, DIM), lambda b, r: (b * grid_r + r, 0)),
                  pl.BlockSpec((n, DIM), lambda b, r: (b, 0))],
        out_specs=pl.BlockSpec((BLK * n // DIM, DIM),
                               lambda b, r: (b * grid_r + r, 0)),
        out_shape=jax.ShapeDtypeStruct((nb * n // DIM, DIM), jnp.float32),
        name="tc_qkT",
    )(q, k)


def _tc_fin(nb, x, aggs):
    grid = nb // BLK
    return pl.pallas_call(
        _tc_fin_body,
        grid=(grid,),
        in_specs=[_row_spec(BLK, DIM)] * 2,
        out_specs=_row_spec(BLK, DIM),
        out_shape=jax.ShapeDtypeStruct((nb, DIM), jnp.float32),
        name="tc_fin",
    )(x, aggs)


def kernel(src_tokens, padded_coordinates, src_distance, src_edge_type,
           edge_index, params):
    b, n = src_tokens.shape
    nb = b * n
    num_layers = len(params['layers'])

    tok = src_tokens.reshape(nb, 1).astype(jnp.int32)
    coords = padded_coordinates.reshape(nb, 3).astype(jnp.float32)
    rows = edge_index[0].astype(jnp.int32)
    cols = edge_index[1].astype(jnp.int32)
    e = rows.shape[0]
    epad = ((e + 4 * CHUNK + 7) // 8) * 8 + 8
    rows_p = jnp.concatenate(
        [rows, jnp.full((epad - e,), nb - 1, jnp.int32)])
    cols_p = jnp.concatenate([cols, jnp.zeros((epad - e,), jnp.int32)])
    bounds = (jnp.arange(NTILES + 1, dtype=jnp.int32) * (nb // NTILES))
    off = jnp.searchsorted(rows, bounds, side='left').astype(jnp.int32)
    off48 = jnp.concatenate([off, jnp.full((15,), e, jnp.int32)])

    embed = params['embed']
    vwT = params['vec_w'].T
    vecb = params['vec_b'].reshape(1, DIM)

    def layer_weights(lp):
        kvw = lp['kv_w']
        kvb = lp['kv_b']
        return (lp['q_w'].T, lp['q_b'].reshape(1, DIM),
                kvw[0:DIM].T, kvb[0:DIM].reshape(1, DIM),
                kvw[DIM:2 * DIM].T, kvb[DIM:2 * DIM].reshape(1, DIM),
                kvw[2 * DIM:].T, kvb[2 * DIM:].reshape(1, DIM))

    lws = [layer_weights(lp) for lp in params['layers']]
    sc_edge = _make_sc_edge(nb, n)

    x, v, q, k, sp = _tc_init(nb, tok, coords, embed, vwT, vecb, lws[0])
    for l in range(num_layers):
        w = _tc_w(nb, n, q, k)
        aggs_f, aggv_f = sc_edge(w.reshape(nb * n), sp, rows_p, cols_p,
                                 off48)
        aggs = aggs_f.reshape(nb, DIM)
        aggv = aggv_f.reshape(nb, DIM)
        if l + 1 < num_layers:
            lp = params['layers'][l]
            x, v, q, k, sp = _tc_mid(
                nb, x, v, aggs, aggv, lp['vm_w'].T,
                lp['vm_b'].reshape(1, DIM), lws[l + 1])
        else:
            x = _tc_fin(nb, x, aggs)

    encoder_rep = x.reshape(b, n, DIM)
    padding_mask = src_tokens == 0
    return encoder_rep, padding_mask
